# baseline (speedup 1.0000x reference)
import functools

import jax
import jax.numpy as jnp
from jax import lax
from jax.experimental import pallas as pl
from jax.experimental.pallas import tpu as pltpu
from jax.experimental.pallas import tpu_sc as plsc

VOCAB = 1000000
MAXLEN = 200
EMBED = 64
BATCH = 1024

N_ROWS = BATCH * MAXLEN
NW = 32
ROWS_PER_W = N_ROWS // NW
CHUNK = 128
NCHUNK = ROWS_PER_W // CHUNK
LANES = 16
WIDE = 2 * EMBED
NBLK = VOCAB // CHUNK
BLK_PER_W = NBLK // NW
NTAIL = VOCAB - NBLK * CHUNK


GRP = 3
GCOLS = GRP * CHUNK
NGRP = NBLK // GRP
GRP_PER_W = NGRP // NW


GPITCH = GCOLS + 1


def _transpose_group(wide_v, trows_v):
    iota = lax.iota(jnp.int32, LANES)
    f_idx = [iota + f0 for f0 in range(0, EMBED, LANES)]

    def tbody(t, carry):
        t_splat = jnp.full((LANES,), t, jnp.int32)
        row = jnp.right_shift(t, 1)
        half = jnp.bitwise_and(t, 1) * EMBED
        vs = [plsc.load_gather(wide_v, [f_idx[j], t_splat])
              for j in range(EMBED // LANES)]
        for j in range(EMBED // LANES):
            trows_v[row, pl.ds(half + j * LANES, LANES)] = vs[j]
        return carry

    lax.fori_loop(0, GCOLS, tbody, 0, unroll=2)


def _make_relayout_kernel():
    mesh = plsc.VectorSubcoreMesh(core_axis_name="c", subcore_axis_name="s")

    @functools.partial(
        pl.kernel,
        mesh=mesh,
        out_type=jax.ShapeDtypeStruct((VOCAB // 2, WIDE), jnp.float32),
        scratch_types=[
            pltpu.VMEM((EMBED, GPITCH), jnp.float32),
            pltpu.VMEM((EMBED, GPITCH), jnp.float32),
            pltpu.VMEM((GCOLS // 2, WIDE), jnp.float32),
            pltpu.VMEM((GCOLS // 2, WIDE), jnp.float32),
            pltpu.SemaphoreType.DMA,
            pltpu.SemaphoreType.DMA,
            pltpu.SemaphoreType.DMA,
            pltpu.SemaphoreType.DMA,
        ],
        compiler_params=pltpu.CompilerParams(use_tc_tiling_on_sc=True,
                                             needs_layout_passes=False),
    )
    def k(tokt_hbm, tail_hbm, out_hbm, in0, in1, tr0, tr1,
          si0, si1, so0, so1):
        wid = lax.axis_index("s") * 2 + lax.axis_index("c")
        extra = NGRP - GRP_PER_W * NW
        ngrp = GRP_PER_W + jnp.where(wid < extra, 1, 0)
        base = wid * GRP_PER_W + jnp.minimum(wid, extra)

        def load_group(g, buf, sem):
            for fb in range(EMBED // 8):
                pltpu.async_copy(
                    tokt_hbm.at[pl.ds(fb * 8, 8), pl.ds(g * GCOLS, GCOLS)],
                    buf.at[pl.ds(fb * 8, 8), pl.ds(0, GCOLS)], sem)

        def wait_group(g, buf, sem):
            for fb in range(EMBED // 8):
                pltpu.make_async_copy(
                    tokt_hbm.at[pl.ds(fb * 8, 8), pl.ds(g * GCOLS, GCOLS)],
                    buf.at[pl.ds(fb * 8, 8), pl.ds(0, GCOLS)], sem).wait()

        def dst_at(g):
            return out_hbm.at[pl.ds(g * (GCOLS // 2), GCOLS // 2),
                              pl.ds(0, WIDE)]

        load_group(base, in0, si0)

        ins = (in0, in1)
        trs = (tr0, tr1)
        sis = (si0, si1)
        sos = (so0, so1)

        def body(i, carry):
            for p in range(2):
                ordn = 2 * i + p
                cur = base + ordn

                @pl.when(ordn < ngrp)
                def _():
                    wait_group(cur, ins[p], sis[p])

                    @pl.when(ordn + 1 < ngrp)
                    def _():
                        load_group(cur + 1, ins[1 - p], sis[1 - p])

                    @pl.when(ordn >= 2)
                    def _():
                        pltpu.make_async_copy(trs[p], dst_at(cur - 2),
                                              sos[p]).wait()

                    _transpose_group(ins[p], trs[p])
                    pltpu.async_copy(trs[p], dst_at(cur), sos[p])
            return carry

        niter = (GRP_PER_W + 1 + 1) // 2 + 1
        lax.fori_loop(0, niter, body, 0)

        for p in range(2):
            lp = jnp.where(jnp.bitwise_and(ngrp - 1, 1) == p,
                           ngrp - 1, ngrp - 2)
            pltpu.make_async_copy(trs[p], dst_at(base + lp), sos[p]).wait()

        @pl.when(wid == NW - 1)
        def _():
            pltpu.sync_copy(tail_hbm, tr0.at[pl.ds(0, NTAIL // 2), :])
            pltpu.sync_copy(tr0.at[pl.ds(0, NTAIL // 2), :],
                            out_hbm.at[pl.ds(NBLK * EMBED, NTAIL // 2),
                                       pl.ds(0, WIDE)])

    return k


def _add_pos_select(wide, out_v, idx_v, pos_v, c, off):
    nj = EMBED // LANES

    def gbody(g, carry):
        r0 = g * LANES
        parv = jnp.bitwise_and(idx_v[pl.ds(c * CHUNK + r0, LANES)], 1) * EMBED
        for rr in range(LANES):
            r = r0 + rr
            p = off + r
            p = jnp.where(p >= MAXLEN, p - MAXLEN, p)
            pbase = p * EMBED
            sbase = parv[rr]
            obase = r * EMBED
            srcs = [wide[r, pl.ds(sbase + j * LANES, LANES)] for j in range(nj)]
            poss = [pos_v[pl.ds(pbase + j * LANES, LANES)] for j in range(nj)]
            for j in range(nj):
                out_v[pl.ds(obase + j * LANES, LANES)] = srcs[j] + poss[j]
        return carry
    lax.fori_loop(0, CHUNK // LANES, gbody, 0)


def _make_lookup_kernel():
    mesh = plsc.VectorSubcoreMesh(core_axis_name="c", subcore_axis_name="s")

    @functools.partial(
        pl.kernel,
        mesh=mesh,
        out_type=jax.ShapeDtypeStruct((N_ROWS * EMBED,), jnp.float32),
        scratch_types=[
            pltpu.VMEM((ROWS_PER_W + LANES,), jnp.int32),
            pltpu.VMEM((ROWS_PER_W,), jnp.int32),
            pltpu.VMEM((MAXLEN * EMBED,), jnp.float32),
            pltpu.VMEM((CHUNK, WIDE), jnp.float32),
            pltpu.VMEM((CHUNK, WIDE), jnp.float32),
            pltpu.VMEM((CHUNK * EMBED,), jnp.float32),
            pltpu.VMEM((CHUNK * EMBED,), jnp.float32),
            pltpu.SemaphoreType.DMA,
            pltpu.SemaphoreType.DMA,
            pltpu.SemaphoreType.DMA,
            pltpu.SemaphoreType.DMA,
        ],
        compiler_params=pltpu.CompilerParams(use_tc_tiling_on_sc=True,
                                             needs_layout_passes=False),
    )
    def k(x_hbm, tok_hbm, pos_hbm, out_hbm, idx_v, half_v, pos_v,
          rows0, rows1, outv0, outv1, sem0, sem1, oso0, oso1):
        wid = lax.axis_index("s") * 2 + lax.axis_index("c")
        pltpu.sync_copy(x_hbm.at[pl.ds(wid * ROWS_PER_W, ROWS_PER_W)],
                        idx_v.at[pl.ds(0, ROWS_PER_W)])
        pltpu.sync_copy(pos_hbm, pos_v)

        def hbody(i, carry):
            sl = pl.ds(i * LANES, LANES)
            half_v[sl] = jnp.right_shift(idx_v[sl], 1)
            return carry
        lax.fori_loop(0, ROWS_PER_W // LANES, hbody, 0, unroll=4)

        def idx_at(c):
            return half_v.at[pl.ds(c * CHUNK, CHUNK)]

        pltpu.async_copy(tok_hbm.at[idx_at(0)], rows0, sem0)
        out_base = wid * ROWS_PER_W * EMBED
        outs = (outv0, outv1)
        osos = (oso0, oso1)
        rows = (rows0, rows1)
        sems = (sem0, sem1)

        def out_at(c):
            return out_hbm.at[pl.ds(out_base + c * CHUNK * EMBED,
                                    CHUNK * EMBED)]

        def gbody(g, carry):
            for b in range(2):
                c = 2 * g + b
                pltpu.make_async_copy(tok_hbm.at[idx_at(c)], rows[b],
                                      sems[b]).wait()

                @pl.when(c < NCHUNK - 1)
                def _():
                    pltpu.async_copy(tok_hbm.at[idx_at(c + 1)], rows[1 - b],
                                     sems[1 - b])

                @pl.when(c >= 2)
                def _():
                    pltpu.make_async_copy(outs[b], out_at(c - 2),
                                          osos[b]).wait()

                _add_pos_select(rows[b], outs[b], idx_v, pos_v, c,
                                lax.rem(c * CHUNK, MAXLEN))
                pltpu.async_copy(outs[b], out_at(c), osos[b])
            return carry

        lax.fori_loop(0, NCHUNK // 2, gbody, 0)

        for b in range(2):
            pltpu.make_async_copy(outs[b], out_at(NCHUNK - 2 + b),
                                  osos[b]).wait()

    return k


_relayout = _make_relayout_kernel()
_lookup = _make_lookup_kernel()


def kernel(x, token_table, pos_table):
    x_flat = x.astype(jnp.int32).reshape(N_ROWS)
    tokt = token_table.T
    tail = token_table[NBLK * CHUNK:].reshape(NTAIL // 2, WIDE)
    pos_flat = pos_table.reshape(MAXLEN * EMBED)
    tok2 = _relayout(tokt, tail)
    out = _lookup(x_flat, tok2, pos_flat)
    return out.reshape(BATCH, MAXLEN, EMBED)


# speedup vs baseline: 1.8010x; 1.8010x over previous
import functools

import jax
import jax.numpy as jnp
from jax import lax
from jax.experimental import pallas as pl
from jax.experimental.pallas import tpu as pltpu
from jax.experimental.pallas import tpu_sc as plsc

VOCAB = 1000000
MAXLEN = 200
EMBED = 64
BATCH = 1024

N_ROWS = BATCH * MAXLEN
NW = 32
ROWS_PER_W = N_ROWS // NW
CHUNK = 128
NCHUNK = ROWS_PER_W // CHUNK
LANES = 16
WIDE = 2 * EMBED
NBLK = VOCAB // CHUNK
BLK_PER_W = NBLK // NW
NTAIL = VOCAB - NBLK * CHUNK


GRP = 3
GCOLS = GRP * CHUNK
NGRP = NBLK // GRP
GRP_PER_W = NGRP // NW


GPITCH = GCOLS + 1


def _transpose_group(wide_v, trows_v):
    iota = lax.iota(jnp.int32, LANES)
    f_idx = [iota + f0 for f0 in range(0, EMBED, LANES)]

    def tbody(t, carry):
        t_splat = jnp.full((LANES,), t, jnp.int32)
        row = jnp.right_shift(t, 1)
        half = jnp.bitwise_and(t, 1) * EMBED
        vs = [plsc.load_gather(wide_v, [f_idx[j], t_splat])
              for j in range(EMBED // LANES)]
        for j in range(EMBED // LANES):
            trows_v[row, pl.ds(half + j * LANES, LANES)] = vs[j]
        return carry

    lax.fori_loop(0, GCOLS, tbody, 0, unroll=2)


def _make_relayout_kernel():
    mesh = plsc.VectorSubcoreMesh(core_axis_name="c", subcore_axis_name="s")

    @functools.partial(
        pl.kernel,
        mesh=mesh,
        out_type=jax.ShapeDtypeStruct((VOCAB // 2, WIDE), jnp.float32),
        scratch_types=[
            pltpu.VMEM((EMBED, GPITCH), jnp.float32),
            pltpu.VMEM((EMBED, GPITCH), jnp.float32),
            pltpu.VMEM((GCOLS // 2, WIDE), jnp.float32),
            pltpu.VMEM((GCOLS // 2, WIDE), jnp.float32),
            pltpu.SemaphoreType.DMA,
            pltpu.SemaphoreType.DMA,
            pltpu.SemaphoreType.DMA,
            pltpu.SemaphoreType.DMA,
        ],
        compiler_params=pltpu.CompilerParams(use_tc_tiling_on_sc=True,
                                             needs_layout_passes=False),
    )
    def k(tokt_hbm, tail_hbm, out_hbm, in0, in1, tr0, tr1,
          si0, si1, so0, so1):
        wid = lax.axis_index("s") * 2 + lax.axis_index("c")
        extra = NGRP - GRP_PER_W * NW
        ngrp = GRP_PER_W + jnp.where(wid < extra, 1, 0)
        base = wid * GRP_PER_W + jnp.minimum(wid, extra)

        def load_group(g, buf, sem):
            for fb in range(EMBED // 8):
                pltpu.async_copy(
                    tokt_hbm.at[pl.ds(fb * 8, 8), pl.ds(g * GCOLS, GCOLS)],
                    buf.at[pl.ds(fb * 8, 8), pl.ds(0, GCOLS)], sem)

        def wait_group(g, buf, sem):
            for fb in range(EMBED // 8):
                pltpu.make_async_copy(
                    tokt_hbm.at[pl.ds(fb * 8, 8), pl.ds(g * GCOLS, GCOLS)],
                    buf.at[pl.ds(fb * 8, 8), pl.ds(0, GCOLS)], sem).wait()

        def dst_at(g):
            return out_hbm.at[pl.ds(g * (GCOLS // 2), GCOLS // 2),
                              pl.ds(0, WIDE)]

        load_group(base, in0, si0)

        ins = (in0, in1)
        trs = (tr0, tr1)
        sis = (si0, si1)
        sos = (so0, so1)

        def body(i, carry):
            for p in range(2):
                ordn = 2 * i + p
                cur = base + ordn

                @pl.when(ordn < ngrp)
                def _():
                    wait_group(cur, ins[p], sis[p])

                    @pl.when(ordn + 1 < ngrp)
                    def _():
                        load_group(cur + 1, ins[1 - p], sis[1 - p])

                    @pl.when(ordn >= 2)
                    def _():
                        pltpu.make_async_copy(trs[p], dst_at(cur - 2),
                                              sos[p]).wait()

                    _transpose_group(ins[p], trs[p])
                    pltpu.async_copy(trs[p], dst_at(cur), sos[p])
            return carry

        niter = (GRP_PER_W + 1 + 1) // 2 + 1
        lax.fori_loop(0, niter, body, 0)

        for p in range(2):
            lp = jnp.where(jnp.bitwise_and(ngrp - 1, 1) == p,
                           ngrp - 1, ngrp - 2)
            pltpu.make_async_copy(trs[p], dst_at(base + lp), sos[p]).wait()

        @pl.when(wid == NW - 1)
        def _():
            pltpu.sync_copy(tail_hbm, tr0.at[pl.ds(0, NTAIL // 2), :])
            pltpu.sync_copy(tr0.at[pl.ds(0, NTAIL // 2), :],
                            out_hbm.at[pl.ds(NBLK * EMBED, NTAIL // 2),
                                       pl.ds(0, WIDE)])

    return k


def _add_pos_select(wide, out_v, idx_v, pos_v, c, off):
    nj = EMBED // LANES

    def gbody(g, carry):
        r0 = g * LANES
        parv = jnp.bitwise_and(idx_v[pl.ds(c * CHUNK + r0, LANES)], 1) * EMBED
        for rr in range(LANES):
            r = r0 + rr
            p = off + r
            p = jnp.where(p >= MAXLEN, p - MAXLEN, p)
            pbase = p * EMBED
            sbase = parv[rr]
            obase = r * EMBED
            srcs = [wide[r, pl.ds(sbase + j * LANES, LANES)] for j in range(nj)]
            poss = [pos_v[pl.ds(pbase + j * LANES, LANES)] for j in range(nj)]
            for j in range(nj):
                out_v[pl.ds(obase + j * LANES, LANES)] = srcs[j] + poss[j]
        return carry
    lax.fori_loop(0, CHUNK // LANES, gbody, 0)


def _make_lookup_kernel():
    mesh = plsc.VectorSubcoreMesh(core_axis_name="c", subcore_axis_name="s")

    @functools.partial(
        pl.kernel,
        mesh=mesh,
        out_type=jax.ShapeDtypeStruct((N_ROWS * EMBED,), jnp.float32),
        scratch_types=[
            pltpu.VMEM((ROWS_PER_W + LANES,), jnp.int32),
            pltpu.VMEM((ROWS_PER_W,), jnp.int32),
            pltpu.VMEM((MAXLEN * EMBED,), jnp.float32),
            pltpu.VMEM((CHUNK, WIDE), jnp.float32),
            pltpu.VMEM((CHUNK, WIDE), jnp.float32),
            pltpu.VMEM((CHUNK * EMBED,), jnp.float32),
            pltpu.VMEM((CHUNK * EMBED,), jnp.float32),
            pltpu.SemaphoreType.DMA,
            pltpu.SemaphoreType.DMA,
            pltpu.SemaphoreType.DMA,
            pltpu.SemaphoreType.DMA,
        ],
        compiler_params=pltpu.CompilerParams(use_tc_tiling_on_sc=True,
                                             needs_layout_passes=False),
    )
    def k(x_hbm, tok_hbm, pos_hbm, out_hbm, idx_v, half_v, pos_v,
          rows0, rows1, outv0, outv1, sem0, sem1, oso0, oso1):
        wid = lax.axis_index("s") * 2 + lax.axis_index("c")
        pltpu.sync_copy(x_hbm.at[pl.ds(wid * ROWS_PER_W, ROWS_PER_W)],
                        idx_v.at[pl.ds(0, ROWS_PER_W)])
        pltpu.sync_copy(pos_hbm, pos_v)

        def hbody(i, carry):
            sl = pl.ds(i * LANES, LANES)
            half_v[sl] = jnp.right_shift(idx_v[sl], 1)
            return carry
        lax.fori_loop(0, ROWS_PER_W // LANES, hbody, 0, unroll=4)

        def idx_at(c):
            return half_v.at[pl.ds(c * CHUNK, CHUNK)]

        pltpu.async_copy(tok_hbm.at[idx_at(0)], rows0, sem0)
        out_base = wid * ROWS_PER_W * EMBED
        outs = (outv0, outv1)
        osos = (oso0, oso1)
        rows = (rows0, rows1)
        sems = (sem0, sem1)

        def out_at(c):
            return out_hbm.at[pl.ds(out_base + c * CHUNK * EMBED,
                                    CHUNK * EMBED)]

        def gbody(g, carry):
            for b in range(2):
                c = 2 * g + b
                pltpu.make_async_copy(tok_hbm.at[idx_at(c)], rows[b],
                                      sems[b]).wait()

                @pl.when(c < NCHUNK - 1)
                def _():
                    pltpu.async_copy(tok_hbm.at[idx_at(c + 1)], rows[1 - b],
                                     sems[1 - b])

                @pl.when(c >= 2)
                def _():
                    pltpu.make_async_copy(outs[b], out_at(c - 2),
                                          osos[b]).wait()

                _add_pos_select(rows[b], outs[b], idx_v, pos_v, c,
                                lax.rem(c * CHUNK, MAXLEN))
                pltpu.async_copy(outs[b], out_at(c), osos[b])
            return carry

        lax.fori_loop(0, NCHUNK // 2, gbody, 0)

        for b in range(2):
            pltpu.make_async_copy(outs[b], out_at(NCHUNK - 2 + b),
                                  osos[b]).wait()

    return k


_lookup = _make_lookup_kernel()


def kernel(x, token_table, pos_table):
    x_flat = x.astype(jnp.int32).reshape(N_ROWS)
    tok2 = token_table.reshape(VOCAB // 2, WIDE)
    pos_flat = pos_table.reshape(MAXLEN * EMBED)
    out = _lookup(x_flat, tok2, pos_flat)
    return out.reshape(BATCH, MAXLEN, EMBED)


# speedup vs baseline: 1.8097x; 1.0048x over previous
import functools

import jax
import jax.numpy as jnp
from jax import lax
from jax.experimental import pallas as pl
from jax.experimental.pallas import tpu as pltpu
from jax.experimental.pallas import tpu_sc as plsc

VOCAB = 1000000
MAXLEN = 200
EMBED = 64
BATCH = 1024

N_ROWS = BATCH * MAXLEN
NW = 32
ROWS_PER_W = N_ROWS // NW
CHUNK = 128
NCHUNK = ROWS_PER_W // CHUNK
LANES = 16
WIDE = 2 * EMBED


def _add_pos_select(wide, out_v, idx_v, pos_v, c, off):
    nj = EMBED // LANES

    def gbody(g, carry):
        r0 = g * LANES
        parv = jnp.bitwise_and(idx_v[pl.ds(c * CHUNK + r0, LANES)], 1) * EMBED
        for rr in range(LANES):
            r = r0 + rr
            p = off + r
            p = jnp.where(p >= MAXLEN, p - MAXLEN, p)
            pbase = p * EMBED
            sbase = parv[rr]
            obase = r * EMBED
            srcs = [wide[r, pl.ds(sbase + j * LANES, LANES)] for j in range(nj)]
            poss = [pos_v[pl.ds(pbase + j * LANES, LANES)] for j in range(nj)]
            for j in range(nj):
                out_v[pl.ds(obase + j * LANES, LANES)] = srcs[j] + poss[j]
        return carry
    lax.fori_loop(0, CHUNK // LANES, gbody, 0)


def _make_lookup_kernel():
    mesh = plsc.VectorSubcoreMesh(core_axis_name="c", subcore_axis_name="s")

    @functools.partial(
        pl.kernel,
        mesh=mesh,
        out_type=jax.ShapeDtypeStruct((N_ROWS * EMBED,), jnp.float32),
        scratch_types=[
            pltpu.VMEM((ROWS_PER_W + LANES,), jnp.int32),
            pltpu.VMEM((ROWS_PER_W,), jnp.int32),
            pltpu.VMEM((MAXLEN * EMBED,), jnp.float32),
            pltpu.VMEM((CHUNK, WIDE), jnp.float32),
            pltpu.VMEM((CHUNK, WIDE), jnp.float32),
            pltpu.VMEM((CHUNK * EMBED,), jnp.float32),
            pltpu.VMEM((CHUNK * EMBED,), jnp.float32),
            pltpu.SemaphoreType.DMA,
            pltpu.SemaphoreType.DMA,
            pltpu.SemaphoreType.DMA,
            pltpu.SemaphoreType.DMA,
        ],
        compiler_params=pltpu.CompilerParams(use_tc_tiling_on_sc=True,
                                             needs_layout_passes=False),
    )
    def k(x_hbm, tok_hbm, pos_hbm, out_hbm, idx_v, half_v, pos_v,
          rows0, rows1, outv0, outv1, sem0, sem1, oso0, oso1):
        wid = lax.axis_index("s") * 2 + lax.axis_index("c")
        pltpu.sync_copy(x_hbm.at[pl.ds(wid * ROWS_PER_W, ROWS_PER_W)],
                        idx_v.at[pl.ds(0, ROWS_PER_W)])
        pltpu.sync_copy(pos_hbm, pos_v)

        def hbody(i, carry):
            sl = pl.ds(i * LANES, LANES)
            half_v[sl] = jnp.right_shift(idx_v[sl], 1)
            return carry
        lax.fori_loop(0, ROWS_PER_W // LANES, hbody, 0, unroll=4)

        def idx_at(c):
            return half_v.at[pl.ds(c * CHUNK, CHUNK)]

        pltpu.async_copy(tok_hbm.at[idx_at(0)], rows0, sem0)
        out_base = wid * ROWS_PER_W * EMBED
        outs = (outv0, outv1)
        osos = (oso0, oso1)
        rows = (rows0, rows1)
        sems = (sem0, sem1)

        def out_at(c):
            return out_hbm.at[pl.ds(out_base + c * CHUNK * EMBED,
                                    CHUNK * EMBED)]

        def gbody(g, carry):
            for b in range(2):
                c = 2 * g + b
                pltpu.make_async_copy(tok_hbm.at[idx_at(c)], rows[b],
                                      sems[b]).wait()

                @pl.when(c < NCHUNK - 1)
                def _():
                    pltpu.async_copy(tok_hbm.at[idx_at(c + 1)], rows[1 - b],
                                     sems[1 - b])

                @pl.when(c >= 2)
                def _():
                    pltpu.make_async_copy(outs[b], out_at(c - 2),
                                          osos[b]).wait()

                _add_pos_select(rows[b], outs[b], idx_v, pos_v, c,
                                lax.rem(c * CHUNK, MAXLEN))
                pltpu.async_copy(outs[b], out_at(c), osos[b])
            return carry

        lax.fori_loop(0, NCHUNK // 2, gbody, 0)

        for b in range(2):
            pltpu.make_async_copy(outs[b], out_at(NCHUNK - 2 + b),
                                  osos[b]).wait()

    return k


_lookup = _make_lookup_kernel()


def kernel(x, token_table, pos_table):
    x_flat = x.astype(jnp.int32).reshape(N_ROWS)
    tok2 = token_table.reshape(VOCAB // 2, WIDE)
    pos_flat = pos_table.reshape(MAXLEN * EMBED)
    out = _lookup(x_flat, tok2, pos_flat)
    return out.reshape(BATCH, MAXLEN, EMBED)


# speedup vs baseline: 1.9657x; 1.0862x over previous
import functools

import jax
import jax.numpy as jnp
from jax import lax
from jax.experimental import pallas as pl
from jax.experimental.pallas import tpu as pltpu
from jax.experimental.pallas import tpu_sc as plsc

VOCAB = 1000000
MAXLEN = 200
EMBED = 64
BATCH = 1024

N_ROWS = BATCH * MAXLEN
NW = 32
ROWS_PER_W = N_ROWS // NW
CHUNK = 128
NCHUNK = ROWS_PER_W // CHUNK
LANES = 16
WIDE = 2 * EMBED


def _add_pos_select(wide, out_v, idx_v, pos_v, c, off):
    nj = EMBED // LANES

    def gbody(g, carry):
        r0 = g * LANES
        parv = jnp.bitwise_and(idx_v[pl.ds(c * CHUNK + r0, LANES)], 1) * EMBED
        for rr in range(LANES):
            r = r0 + rr
            p = off + r
            p = jnp.where(p >= MAXLEN, p - MAXLEN, p)
            pbase = p * EMBED
            sbase = parv[rr]
            srcs = [wide[r, pl.ds(sbase + j * LANES, LANES)] for j in range(nj)]
            poss = [pos_v[pl.ds(pbase + j * LANES, LANES)] for j in range(nj)]
            for j in range(nj):
                out_v[r, pl.ds(j * LANES, LANES)] = srcs[j] + poss[j]
        return carry
    lax.fori_loop(0, CHUNK // LANES, gbody, 0)


def _make_lookup_kernel():
    mesh = plsc.VectorSubcoreMesh(core_axis_name="c", subcore_axis_name="s")

    @functools.partial(
        pl.kernel,
        mesh=mesh,
        out_type=jax.ShapeDtypeStruct((N_ROWS, WIDE), jnp.float32),
        scratch_types=[
            pltpu.VMEM((ROWS_PER_W + LANES,), jnp.int32),
            pltpu.VMEM((ROWS_PER_W,), jnp.int32),
            pltpu.VMEM((MAXLEN * EMBED,), jnp.float32),
            pltpu.VMEM((CHUNK, WIDE), jnp.float32),
            pltpu.VMEM((CHUNK, WIDE), jnp.float32),
            pltpu.VMEM((CHUNK, WIDE), jnp.float32),
            pltpu.VMEM((CHUNK, WIDE), jnp.float32),
            pltpu.SemaphoreType.DMA,
            pltpu.SemaphoreType.DMA,
            pltpu.SemaphoreType.DMA,
            pltpu.SemaphoreType.DMA,
        ],
        compiler_params=pltpu.CompilerParams(use_tc_tiling_on_sc=True,
                                             needs_layout_passes=False),
    )
    def k(x_hbm, tok_hbm, pos_hbm, out_hbm, idx_v, half_v, pos_v,
          rows0, rows1, outv0, outv1, sem0, sem1, oso0, oso1):
        wid = lax.axis_index("s") * 2 + lax.axis_index("c")
        pltpu.sync_copy(x_hbm.at[pl.ds(wid * ROWS_PER_W, ROWS_PER_W)],
                        idx_v.at[pl.ds(0, ROWS_PER_W)])
        pltpu.sync_copy(pos_hbm, pos_v)

        def hbody(i, carry):
            sl = pl.ds(i * LANES, LANES)
            half_v[sl] = jnp.right_shift(idx_v[sl], 1)
            return carry
        lax.fori_loop(0, ROWS_PER_W // LANES, hbody, 0, unroll=4)

        def idx_at(c):
            return half_v.at[pl.ds(c * CHUNK, CHUNK)]

        pltpu.async_copy(tok_hbm.at[idx_at(0)], rows0, sem0)
        out_base = wid * ROWS_PER_W
        outs = (outv0, outv1)
        osos = (oso0, oso1)
        rows = (rows0, rows1)
        sems = (sem0, sem1)

        def out_at(c):
            return out_hbm.at[pl.ds(out_base + c * CHUNK, CHUNK),
                              pl.ds(0, WIDE)]

        def gbody(g, carry):
            for b in range(2):
                c = 2 * g + b
                pltpu.make_async_copy(tok_hbm.at[idx_at(c)], rows[b],
                                      sems[b]).wait()

                @pl.when(c < NCHUNK - 1)
                def _():
                    pltpu.async_copy(tok_hbm.at[idx_at(c + 1)], rows[1 - b],
                                     sems[1 - b])

                @pl.when(c >= 2)
                def _():
                    pltpu.make_async_copy(outs[b], out_at(c - 2),
                                          osos[b]).wait()

                _add_pos_select(rows[b], outs[b], idx_v, pos_v, c,
                                lax.rem(c * CHUNK, MAXLEN))
                pltpu.async_copy(outs[b], out_at(c), osos[b])
            return carry

        lax.fori_loop(0, NCHUNK // 2, gbody, 0)

        for b in range(2):
            pltpu.make_async_copy(outs[b], out_at(NCHUNK - 2 + b),
                                  osos[b]).wait()

    return k


_lookup = _make_lookup_kernel()


def kernel(x, token_table, pos_table):
    x_flat = x.astype(jnp.int32).reshape(N_ROWS)
    tok2 = token_table.reshape(VOCAB // 2, WIDE)
    pos_flat = pos_table.reshape(MAXLEN * EMBED)
    out = _lookup(x_flat, tok2, pos_flat)
    return out[:, :EMBED].reshape(BATCH, MAXLEN, EMBED)
